# Initial kernel scaffold; baseline (speedup 1.0000x reference)
import functools
import jax
import jax.numpy as jnp
from jax import lax
from jax.experimental import pallas as pl
from jax.experimental.pallas import tpu as pltpu
from jax.experimental.pallas import tpu_sc as plsc

T = 4
K = 16
CH = 128
N = 10000
NW = 32
NPT = 320
NPAD = NW * NPT
BN = 4
NBLK = NPT // BN
EPB = BN * T * K
NBT = 512


def _table_body(m_ref, b1_ref, b2_ref, x_ref, t1_ref, t2_ref):
    xb = x_ref[0]
    m = m_ref[...]
    z = lax.dot_general(xb, m, (((0,), (0,)), ((), ())),
                        preferred_element_type=jnp.float32)
    t1_ref[...] = z + b1_ref[...]
    t2_ref[...] = z[:, 64:128] + b2_ref[...]


def _make_tables(xfp, m, bias1, bias2):
    grid = (T, NPAD // NBT)
    return pl.pallas_call(
        _table_body,
        grid=grid,
        in_specs=[
            pl.BlockSpec((128, 128), lambda t, j: (0, 0)),
            pl.BlockSpec((1, 128), lambda t, j: (0, 0)),
            pl.BlockSpec((1, 64), lambda t, j: (0, 0)),
            pl.BlockSpec((1, 128, NBT), lambda t, j: (t, 0, j)),
        ],
        out_specs=[
            pl.BlockSpec((NBT, 128), lambda t, j: (t * (NPAD // NBT) + j, 0)),
            pl.BlockSpec((NBT, 64), lambda t, j: (t * (NPAD // NBT) + j, 0)),
        ],
        out_shape=[
            jax.ShapeDtypeStruct((T * NPAD, 128), jnp.float32),
            jax.ShapeDtypeStruct((T * NPAD, 64), jnp.float32),
        ],
    )(m, bias1, bias2, xfp)


def _sc_body(t1_hbm, t2_hbm, i1_hbm, i0_hbm, out_hbm,
             idx1_v, idx0_v, rows1_v, rows2_v, out_v, sem):
    wid = lax.axis_index("s") * 2 + lax.axis_index("c")

    def block_fn(blk, carry):
        nb = wid * NPT + blk * BN
        irow = nb >> 1
        pltpu.sync_copy(i1_hbm.at[pl.ds(irow, 2), :], idx1_v)
        pltpu.sync_copy(i0_hbm.at[pl.ds(irow, 2), :], idx0_v)
        cps = [
            pltpu.async_copy(t1_hbm.at[idx1_v.at[0]],
                             rows1_v.at[pl.ds(0, 128), :], sem),
            pltpu.async_copy(t1_hbm.at[idx1_v.at[1]],
                             rows1_v.at[pl.ds(128, 128), :], sem),
            pltpu.async_copy(t2_hbm.at[idx0_v.at[0]],
                             rows2_v.at[pl.ds(0, 128), :], sem),
            pltpu.async_copy(t2_hbm.at[idx0_v.at[1]],
                             rows2_v.at[pl.ds(128, 128), :], sem),
        ]
        for c in cps:
            c.wait()
        for nl in range(BN):
            def k_body(k, smax):
                r0 = nl * (T * K) + k
                sm = list(smax)
                v = [jnp.zeros((16,), jnp.float32)] * 8
                for t in range(T):
                    r = r0 + t * K
                    for cg in range(8):
                        lo = rows1_v[r, pl.ds(cg * 16, 16)]
                        if cg < 4:
                            conv = lo
                        else:
                            conv = rows2_v[r, pl.ds((cg - 4) * 16, 16)] - lo
                        vv = v[cg] + conv
                        spk = vv >= 1.0
                        v[cg] = jnp.where(spk, 0.0, vv)
                        sm[t * 8 + cg] = jnp.where(spk, 1.0, sm[t * 8 + cg])
                return tuple(sm)

            zero = jnp.zeros((16,), jnp.float32)
            smax = lax.fori_loop(0, K, k_body, (zero,) * (T * 8))
            for t in range(T):
                for cg in range(8):
                    out_v[nl, t, pl.ds(cg * 16, 16)] = smax[t * 8 + cg]
        pltpu.sync_copy(out_v, out_hbm.at[pl.ds(nb, BN), :, :])
        return carry

    lax.fori_loop(0, NBLK, block_fn, 0)


_sc_call = functools.partial(
    pl.kernel,
    out_type=jax.ShapeDtypeStruct((NPAD, T, CH), jnp.float32),
    mesh=plsc.VectorSubcoreMesh(core_axis_name="c", subcore_axis_name="s"),
    scratch_types=[
        pltpu.VMEM((2, 128), jnp.int32),
        pltpu.VMEM((2, 128), jnp.int32),
        pltpu.VMEM((EPB, 128), jnp.float32),
        pltpu.VMEM((EPB, 64), jnp.float32),
        pltpu.VMEM((BN, T, CH), jnp.float32),
        pltpu.SemaphoreType.DMA,
    ],
)(_sc_body)


@jax.jit
def kernel(x, edge_index, W, b):
    xf = x[..., 0]
    xfp = jnp.pad(xf, ((0, 0), (0, 0), (0, NPAD - N)))
    Wg = W.reshape(4, 32, 64)
    m = jnp.zeros((128, 128), W.dtype)
    m = m.at[0:64, 0:32].set(Wg[0].T)
    m = m.at[64:128, 32:64].set(Wg[1].T)
    m = m.at[0:64, 64:96].set(Wg[2].T)
    m = m.at[64:128, 96:128].set(Wg[3].T)
    bias1 = jnp.concatenate([b[:64], jnp.zeros((64,), b.dtype)])[None, :]
    bias2 = b[64:128][None, :]
    t1, t2 = _make_tables(xfp, m, bias1, bias2)

    ei = jnp.pad(edge_index, ((0, 0), (0, 0), (0, NPAD - N), (0, 0)))
    shift = (jnp.arange(T, dtype=jnp.int32) * NPAD)[None, :, None, None]
    eis = ei + shift
    i1 = eis[1].transpose(1, 0, 2).reshape(-1, 128)
    i0 = eis[0].transpose(1, 0, 2).reshape(-1, 128)

    out_s = _sc_call(t1, t2, i1, i0)
    out = out_s[:N].transpose(1, 2, 0)
    return out[:, None, :, :, None]


# speedup vs baseline: 8.8507x; 8.8507x over previous
import functools
import numpy as np
import jax
import jax.numpy as jnp
from jax import lax
from jax.experimental import pallas as pl
from jax.experimental.pallas import tpu as pltpu
from jax.experimental.pallas import tpu_sc as plsc

T = 4
K = 16
CH = 128
N = 10000
NW = 32
NPT = 320
NPAD = NW * NPT
SBR = NPAD * K // 128
RPT = NPT * K // 128
E = NPAD * K
RC = 1024


def _sc_body(xf_hbm, i1_hbm, i0_hbm,
             fc0, fc1, fc2, fc3,
             idx1_v, idx0_v, xi_v, xj_v, fc_v, sem):
    fc_out = (fc0, fc1, fc2, fc3)
    wid = lax.axis_index("s") * 2 + lax.axis_index("c")

    def ssb_body(ssb, carry):
        for t in range(T):
            rb = pl.multiple_of(t * SBR + wid * RPT + ssb * 8, 8)
            pltpu.sync_copy(i1_hbm.at[pl.ds(rb, 8), :], idx1_v)
            pltpu.sync_copy(i0_hbm.at[pl.ds(rb, 8), :], idx0_v)
            for j in range(8):
                c1 = pltpu.async_copy(xf_hbm.at[idx1_v.at[j]], xi_v, sem)
                c2 = pltpu.async_copy(xf_hbm.at[idx0_v.at[j]], xj_v, sem)
                c1.wait()
                c2.wait()

                def e_body(e, c):
                    for h in range(8):
                        xia = xi_v[e, pl.ds(h * 16, 16)]
                        da = xj_v[e, pl.ds(h * 16, 16)] - xia
                        pw = plsc.pack(
                            xia, da, format=plsc.PackFormat.INTERLEAVED)
                        fc_v[e, pl.ds(h * 16, 16)] = plsc.bitcast(
                            pw, jnp.int32)
                    return c

                lax.fori_loop(0, 128, e_body, 0)
                base = pl.multiple_of(
                    (wid * NPT + ssb * 64 + j * 8) * K, 128)
                pltpu.sync_copy(fc_v, fc_out[t].at[pl.ds(base, 128), :])
        return carry

    lax.fori_loop(0, NPT // 64, ssb_body, 0)


_sc_call = functools.partial(
    pl.kernel,
    out_type=[jax.ShapeDtypeStruct((E, CH), jnp.int32)] * 4,
    mesh=plsc.VectorSubcoreMesh(core_axis_name="c", subcore_axis_name="s"),
    compiler_params=pltpu.CompilerParams(needs_layout_passes=False),
    scratch_types=[
        pltpu.VMEM((8, 128), jnp.int32),
        pltpu.VMEM((8, 128), jnp.int32),
        pltpu.VMEM((128, 128), jnp.float32),
        pltpu.VMEM((128, 128), jnp.float32),
        pltpu.VMEM((128, 128), jnp.int32),
        pltpu.SemaphoreType.DMA,
    ],
)(_sc_body)


def _tc_body(m_ref, b_ref,
             f0_ref, f1_ref, f2_ref, f3_ref,
             o0_ref, o1_ref, o2_ref, o3_ref):
    mw = m_ref[...]
    bias = b_ref[...]
    fs = (f0_ref, f1_ref, f2_ref, f3_ref)
    os = (o0_ref, o1_ref, o2_ref, o3_ref)
    v = jnp.zeros((RC, CH), jnp.float32)
    for t in range(T):
        feat = pltpu.bitcast(fs[t][...], jnp.bfloat16).reshape(RC, 2 * CH)
        conv = lax.dot_general(feat, mw, (((1,), (0,)), ((), ())),
                               preferred_element_type=jnp.float32) + bias
        v = v + conv
        spk = v >= 1.0
        s = jnp.where(spk, 1.0, 0.0)
        v = jnp.where(spk, 0.0, v)
        m = s.reshape(RC // K, K, CH)
        m = jnp.maximum(m[:, :8], m[:, 8:])
        m = jnp.maximum(m[:, :4], m[:, 4:])
        m = jnp.maximum(m[:, :2], m[:, 2:])
        m = jnp.maximum(m[:, :1], m[:, 1:])
        os[t][...] = m.reshape(RC // K, CH)


def _tc_call(mw, bias, feats):
    grid = (E // RC,)
    mspec = pl.BlockSpec((2 * CH, 128), lambda g: (0, 0))
    bspec = pl.BlockSpec((1, 128), lambda g: (0, 0))
    fspec = pl.BlockSpec((RC, 128), lambda g: (g, 0))
    ospec = pl.BlockSpec((RC // K, 128), lambda g: (g, 0))
    return pl.pallas_call(
        _tc_body,
        grid=grid,
        in_specs=[mspec, bspec] + [fspec] * 4,
        out_specs=[ospec] * 4,
        out_shape=[jax.ShapeDtypeStruct((NPAD, CH), jnp.float32)] * 4,
    )(mw, bias, *feats)


@jax.jit
def kernel(x, edge_index, W, b):
    xf = x[..., 0]
    xfp = jnp.pad(xf.transpose(0, 2, 1), ((0, 0), (0, NPAD - N), (0, 0)))
    xfT = xfp.reshape(T * NPAD, CH)

    ei = jnp.pad(edge_index, ((0, 0), (0, 0), (0, NPAD - N), (0, 0)))
    shift = (jnp.arange(T, dtype=jnp.int32) * NPAD)[None, :, None, None]
    eis = ei + shift
    i1 = eis[1].reshape(T * SBR, 128)
    i0 = eis[0].reshape(T * SBR, 128)

    Wg = W.reshape(4, 32, 64)
    mw = jnp.zeros((256, 128), jnp.float32)
    mw = mw.at[0:64, 0:32].set(Wg[0].T)
    mw = mw.at[64:128, 32:64].set(Wg[1].T)
    mw = mw.at[128:192, 64:96].set(Wg[2].T)
    mw = mw.at[192:256, 96:128].set(Wg[3].T)
    mw = mw.astype(jnp.bfloat16)
    bias = b[None, :]

    feats = _sc_call(xfT, i1, i0)
    outs = _tc_call(mw, bias, feats)
    out = jnp.stack([o[:N] for o in outs])
    return out.transpose(0, 2, 1)[:, None, :, :, None]


# speedup vs baseline: 9.5852x; 1.0830x over previous
import functools
import numpy as np
import jax
import jax.numpy as jnp
from jax import lax
from jax.experimental import pallas as pl
from jax.experimental.pallas import tpu as pltpu
from jax.experimental.pallas import tpu_sc as plsc

T = 4
K = 16
CH = 128
N = 10000
NW = 32
NPT = 320
NPAD = NW * NPT
SBR = NPAD * K // 128
RPT = NPT * K // 128
E = NPAD * K
RC = 1024


def _sc_body(xf_hbm, i1_hbm, i0_hbm,
             fc0, fc1, fc2, fc3,
             idx1_v, idx0_v, xi_v, xj_v, fc_v, semg, semo):
    fc_out = (fc0, fc1, fc2, fc3)
    wid = lax.axis_index("s") * 2 + lax.axis_index("c")

    def ssb_body(ssb, carry):
        for t in range(T):
            rb = pl.multiple_of(t * SBR + wid * RPT + ssb * 8, 8)
            pltpu.sync_copy(i1_hbm.at[pl.ds(rb, 8), :], idx1_v)
            pltpu.sync_copy(i0_hbm.at[pl.ds(rb, 8), :], idx0_v)

            def issue(j):
                b = j % 2
                return (
                    pltpu.async_copy(
                        xf_hbm.at[idx1_v.at[j]], xi_v.at[b], semg),
                    pltpu.async_copy(
                        xf_hbm.at[idx0_v.at[j]], xj_v.at[b], semg),
                )

            gh = [issue(0), None]
            oh = [None, None]
            for j in range(8):
                b = j % 2
                c1, c2 = gh[b]
                c1.wait()
                c2.wait()
                if j < 7:
                    gh[(j + 1) % 2] = issue(j + 1)
                if oh[b] is not None:
                    oh[b].wait()

                def e_body(e, c):
                    for h in range(8):
                        xia = xi_v[b, e, pl.ds(h * 16, 16)]
                        da = xj_v[b, e, pl.ds(h * 16, 16)] - xia
                        pw = plsc.pack(
                            xia, da, format=plsc.PackFormat.INTERLEAVED)
                        fc_v[b, e, pl.ds(h * 16, 16)] = plsc.bitcast(
                            pw, jnp.int32)
                    return c

                lax.fori_loop(0, 128, e_body, 0)
                base = pl.multiple_of(
                    (wid * NPT + ssb * 64 + j * 8) * K, 128)
                oh[b] = pltpu.async_copy(
                    fc_v.at[b], fc_out[t].at[pl.ds(base, 128), :], semo)
            for b in range(2):
                if oh[b] is not None:
                    oh[b].wait()
        return carry

    lax.fori_loop(0, NPT // 64, ssb_body, 0)


_sc_call = functools.partial(
    pl.kernel,
    out_type=[jax.ShapeDtypeStruct((E, CH), jnp.int32)] * 4,
    mesh=plsc.VectorSubcoreMesh(core_axis_name="c", subcore_axis_name="s"),
    compiler_params=pltpu.CompilerParams(needs_layout_passes=False),
    scratch_types=[
        pltpu.VMEM((8, 128), jnp.int32),
        pltpu.VMEM((8, 128), jnp.int32),
        pltpu.VMEM((2, 128, 128), jnp.float32),
        pltpu.VMEM((2, 128, 128), jnp.float32),
        pltpu.VMEM((2, 128, 128), jnp.int32),
        pltpu.SemaphoreType.DMA,
        pltpu.SemaphoreType.DMA,
    ],
)(_sc_body)


def _tc_body(m_ref, b_ref,
             f0_ref, f1_ref, f2_ref, f3_ref,
             o0_ref, o1_ref, o2_ref, o3_ref):
    mw = m_ref[...]
    bias = b_ref[...]
    fs = (f0_ref, f1_ref, f2_ref, f3_ref)
    os = (o0_ref, o1_ref, o2_ref, o3_ref)
    v = jnp.zeros((RC, CH), jnp.float32)
    for t in range(T):
        feat = pltpu.bitcast(fs[t][...], jnp.bfloat16).reshape(RC, 2 * CH)
        conv = lax.dot_general(feat, mw, (((1,), (0,)), ((), ())),
                               preferred_element_type=jnp.float32) + bias
        v = v + conv
        spk = v >= 1.0
        s = jnp.where(spk, 1.0, 0.0)
        v = jnp.where(spk, 0.0, v)
        m = s.reshape(RC // K, K, CH)
        m = jnp.maximum(m[:, :8], m[:, 8:])
        m = jnp.maximum(m[:, :4], m[:, 4:])
        m = jnp.maximum(m[:, :2], m[:, 2:])
        m = jnp.maximum(m[:, :1], m[:, 1:])
        os[t][...] = m.reshape(RC // K, CH)


def _tc_call(mw, bias, feats):
    grid = (E // RC,)
    mspec = pl.BlockSpec((2 * CH, 128), lambda g: (0, 0))
    bspec = pl.BlockSpec((1, 128), lambda g: (0, 0))
    fspec = pl.BlockSpec((RC, 128), lambda g: (g, 0))
    ospec = pl.BlockSpec((RC // K, 128), lambda g: (g, 0))
    return pl.pallas_call(
        _tc_body,
        grid=grid,
        in_specs=[mspec, bspec] + [fspec] * 4,
        out_specs=[ospec] * 4,
        out_shape=[jax.ShapeDtypeStruct((NPAD, CH), jnp.float32)] * 4,
    )(mw, bias, *feats)


@jax.jit
def kernel(x, edge_index, W, b):
    xf = x[..., 0]
    xfp = jnp.pad(xf.transpose(0, 2, 1), ((0, 0), (0, NPAD - N), (0, 0)))
    xfT = xfp.reshape(T * NPAD, CH)

    ei = jnp.pad(edge_index, ((0, 0), (0, 0), (0, NPAD - N), (0, 0)))
    shift = (jnp.arange(T, dtype=jnp.int32) * NPAD)[None, :, None, None]
    eis = ei + shift
    i1 = eis[1].reshape(T * SBR, 128)
    i0 = eis[0].reshape(T * SBR, 128)

    Wg = W.reshape(4, 32, 64)
    mw = jnp.zeros((256, 128), jnp.float32)
    mw = mw.at[0:64, 0:32].set(Wg[0].T)
    mw = mw.at[64:128, 32:64].set(Wg[1].T)
    mw = mw.at[128:192, 64:96].set(Wg[2].T)
    mw = mw.at[192:256, 96:128].set(Wg[3].T)
    mw = mw.astype(jnp.bfloat16)
    bias = b[None, :]

    feats = _sc_call(xfT, i1, i0)
    outs = _tc_call(mw, bias, feats)
    out = jnp.stack([o[:N] for o in outs])
    return out.transpose(0, 2, 1)[:, None, :, :, None]


# speedup vs baseline: 34.8153x; 3.6322x over previous
import functools
import numpy as np
import jax
import jax.numpy as jnp
from jax import lax
from jax.experimental import pallas as pl
from jax.experimental.pallas import tpu as pltpu
from jax.experimental.pallas import tpu_sc as plsc

T = 4
K = 16
CH = 128
N = 10000
NW = 32
NPT = 320
NPAD = NW * NPT
SBR = NPAD * K // 128
RPT = NPT * K // 128
E = NPAD * K
RC = 1024


def _sc_body(xf_hbm, i1_hbm, i0_hbm,
             fc0, fc1, fc2, fc3,
             idx1_v, idx0_v, xi_v, xj_v, fc_v, xfs, semg, semo):
    fc_out = (fc0, fc1, fc2, fc3)
    sid = lax.axis_index("s")
    wid = sid * 2 + lax.axis_index("c")

    for t in range(T):
        @pl.when(sid == 0)
        def _stage():
            pltpu.sync_copy(
                xf_hbm.at[pl.ds(t * NPAD, NPAD), :], xfs)

        plsc.subcore_barrier()

        def ssb_body(ssb, carry):
            rb = pl.multiple_of(t * SBR + wid * RPT + ssb * 8, 8)
            pltpu.sync_copy(i1_hbm.at[pl.ds(rb, 8), :], idx1_v)
            pltpu.sync_copy(i0_hbm.at[pl.ds(rb, 8), :], idx0_v)

            def issue(j):
                b = j % 2
                r, q = j // 2, (j % 2) * 64
                return (
                    pltpu.async_copy(
                        xfs.at[idx1_v.at[r, pl.ds(q, 64)]],
                        xi_v.at[b], semg),
                    pltpu.async_copy(
                        xfs.at[idx0_v.at[r, pl.ds(q, 64)]],
                        xj_v.at[b], semg),
                )

            gh = [issue(0), None]
            oh = None
            for j in range(16):
                b = j % 2
                c1, c2 = gh[b]
                c1.wait()
                c2.wait()
                if j < 15:
                    gh[(j + 1) % 2] = issue(j + 1)
                if oh is not None:
                    oh.wait()

                def e_body(e, c):
                    for h in range(8):
                        xia = xi_v[b, e, pl.ds(h * 16, 16)]
                        da = xj_v[b, e, pl.ds(h * 16, 16)] - xia
                        pw = plsc.pack(
                            xia, da, format=plsc.PackFormat.INTERLEAVED)
                        fc_v[e, pl.ds(h * 16, 16)] = plsc.bitcast(
                            pw, jnp.int32)
                    return c

                lax.fori_loop(0, 64, e_body, 0)
                base = pl.multiple_of(
                    (wid * NPT + ssb * 64 + j * 4) * K, 64)
                oh = pltpu.async_copy(
                    fc_v, fc_out[t].at[pl.ds(base, 64), :], semo)
            if oh is not None:
                oh.wait()
            return carry

        lax.fori_loop(0, NPT // 64, ssb_body, 0)
        plsc.subcore_barrier()


_sc_call = functools.partial(
    pl.kernel,
    out_type=[jax.ShapeDtypeStruct((E, CH), jnp.int32)] * 4,
    mesh=plsc.VectorSubcoreMesh(core_axis_name="c", subcore_axis_name="s"),
    compiler_params=pltpu.CompilerParams(needs_layout_passes=False),
    scratch_types=[
        pltpu.VMEM((8, 128), jnp.int32),
        pltpu.VMEM((8, 128), jnp.int32),
        pltpu.VMEM((2, 64, 128), jnp.float32),
        pltpu.VMEM((2, 64, 128), jnp.float32),
        pltpu.VMEM((64, 128), jnp.int32),
        pltpu.VMEM_SHARED((NPAD, CH), jnp.float32),
        pltpu.SemaphoreType.DMA,
        pltpu.SemaphoreType.DMA,
    ],
)(_sc_body)


def _tc_body(m_ref, b_ref,
             f0_ref, f1_ref, f2_ref, f3_ref,
             o0_ref, o1_ref, o2_ref, o3_ref):
    mw = m_ref[...]
    bias = b_ref[...]
    fs = (f0_ref, f1_ref, f2_ref, f3_ref)
    os = (o0_ref, o1_ref, o2_ref, o3_ref)
    v = jnp.zeros((RC, CH), jnp.float32)
    for t in range(T):
        feat = pltpu.bitcast(fs[t][...], jnp.bfloat16).reshape(RC, 2 * CH)
        conv = lax.dot_general(feat, mw, (((1,), (0,)), ((), ())),
                               preferred_element_type=jnp.float32) + bias
        v = v + conv
        spk = v >= 1.0
        s = jnp.where(spk, 1.0, 0.0)
        v = jnp.where(spk, 0.0, v)
        m = s.reshape(RC // K, K, CH)
        m = jnp.maximum(m[:, :8], m[:, 8:])
        m = jnp.maximum(m[:, :4], m[:, 4:])
        m = jnp.maximum(m[:, :2], m[:, 2:])
        m = jnp.maximum(m[:, :1], m[:, 1:])
        os[t][...] = m.reshape(RC // K, CH)


def _tc_call(mw, bias, feats):
    grid = (E // RC,)
    mspec = pl.BlockSpec((2 * CH, 128), lambda g: (0, 0))
    bspec = pl.BlockSpec((1, 128), lambda g: (0, 0))
    fspec = pl.BlockSpec((RC, 128), lambda g: (g, 0))
    ospec = pl.BlockSpec((RC // K, 128), lambda g: (g, 0))
    return pl.pallas_call(
        _tc_body,
        grid=grid,
        in_specs=[mspec, bspec] + [fspec] * 4,
        out_specs=[ospec] * 4,
        out_shape=[jax.ShapeDtypeStruct((NPAD, CH), jnp.float32)] * 4,
    )(mw, bias, *feats)


@jax.jit
def kernel(x, edge_index, W, b):
    xf = x[..., 0]
    xfp = jnp.pad(xf.transpose(0, 2, 1), ((0, 0), (0, NPAD - N), (0, 0)))
    xfT = xfp.reshape(T * NPAD, CH)

    ei = jnp.pad(edge_index, ((0, 0), (0, 0), (0, NPAD - N), (0, 0)))
    i1 = ei[1].reshape(T * SBR, 128)
    i0 = ei[0].reshape(T * SBR, 128)

    Wg = W.reshape(4, 32, 64)
    mw = jnp.zeros((256, 128), jnp.float32)
    mw = mw.at[0:64, 0:32].set(Wg[0].T)
    mw = mw.at[64:128, 32:64].set(Wg[1].T)
    mw = mw.at[128:192, 64:96].set(Wg[2].T)
    mw = mw.at[192:256, 96:128].set(Wg[3].T)
    mw = mw.astype(jnp.bfloat16)
    bias = b[None, :]

    feats = _sc_call(xfT, i1, i0)
    outs = _tc_call(mw, bias, feats)
    out = jnp.stack([o[:N] for o in outs])
    return out.transpose(0, 2, 1)[:, None, :, :, None]


# speedup vs baseline: 36.9457x; 1.0612x over previous
import functools
import numpy as np
import jax
import jax.numpy as jnp
from jax import lax
from jax.experimental import pallas as pl
from jax.experimental.pallas import tpu as pltpu
from jax.experimental.pallas import tpu_sc as plsc

T = 4
K = 16
CH = 128
N = 10000
NW = 32
NPT = 320
NPAD = NW * NPT
SBR = NPAD * K // 128
RPT = NPT * K // 128
E = NPAD * K
RC = 1024


def _mk_sc_body(ssb0, nssb):
    npt_c = nssb * 64

    def _sc_body(xf_hbm, i1_hbm, i0_hbm,
                 fc0, fc1, fc2, fc3,
                 idx1_v, idx0_v, xi_v, xj_v, fc_v, xfs, semg, semo):
        fc_out = (fc0, fc1, fc2, fc3)
        sid = lax.axis_index("s")
        wid = sid * 2 + lax.axis_index("c")

        for t in range(T):
            @pl.when(sid == 0)
            def _stage():
                pltpu.sync_copy(
                    xf_hbm.at[pl.ds(t * NPAD, NPAD), :], xfs)

            plsc.subcore_barrier()

            def ssb_body(i, carry):
                rb = pl.multiple_of(
                    t * SBR + wid * RPT + (ssb0 + i) * 8, 8)
                pltpu.sync_copy(i1_hbm.at[pl.ds(rb, 8), :], idx1_v)
                pltpu.sync_copy(i0_hbm.at[pl.ds(rb, 8), :], idx0_v)

                def issue(j):
                    b = j % 2
                    r, q = j // 2, (j % 2) * 64
                    return (
                        pltpu.async_copy(
                            xfs.at[idx1_v.at[r, pl.ds(q, 64)]],
                            xi_v.at[b], semg),
                        pltpu.async_copy(
                            xfs.at[idx0_v.at[r, pl.ds(q, 64)]],
                            xj_v.at[b], semg),
                    )

                gh = [issue(0), None]
                oh = None
                for j in range(16):
                    b = j % 2
                    c1, c2 = gh[b]
                    c1.wait()
                    c2.wait()
                    if j < 15:
                        gh[(j + 1) % 2] = issue(j + 1)
                    if oh is not None:
                        oh.wait()

                    def e_body(e, c):
                        for h in range(8):
                            xia = xi_v[b, e, pl.ds(h * 16, 16)]
                            da = xj_v[b, e, pl.ds(h * 16, 16)] - xia
                            pw = plsc.pack(
                                xia, da, format=plsc.PackFormat.INTERLEAVED)
                            fc_v[e, pl.ds(h * 16, 16)] = plsc.bitcast(
                                pw, jnp.int32)
                        return c

                    lax.fori_loop(0, 64, e_body, 0)
                    base = pl.multiple_of(
                        (wid * npt_c + i * 64 + j * 4) * K, 64)
                    oh = pltpu.async_copy(
                        fc_v, fc_out[t].at[pl.ds(base, 64), :], semo)
                if oh is not None:
                    oh.wait()
                return carry

            lax.fori_loop(0, nssb, ssb_body, 0)
            plsc.subcore_barrier()

    return _sc_body


def _mk_sc_call(ssb0, nssb):
    e_c = NW * nssb * 64 * K
    return functools.partial(
        pl.kernel,
        out_type=[jax.ShapeDtypeStruct((e_c, CH), jnp.int32)] * 4,
        mesh=plsc.VectorSubcoreMesh(core_axis_name="c", subcore_axis_name="s"),
        compiler_params=pltpu.CompilerParams(needs_layout_passes=False),
        scratch_types=[
            pltpu.VMEM((8, 128), jnp.int32),
            pltpu.VMEM((8, 128), jnp.int32),
            pltpu.VMEM((2, 64, 128), jnp.float32),
            pltpu.VMEM((2, 64, 128), jnp.float32),
            pltpu.VMEM((64, 128), jnp.int32),
            pltpu.VMEM_SHARED((NPAD, CH), jnp.float32),
            pltpu.SemaphoreType.DMA,
            pltpu.SemaphoreType.DMA,
        ],
    )(_mk_sc_body(ssb0, nssb))


_NSSB_A, _NSSB_B = 3, 2
_sc_call_a = _mk_sc_call(0, _NSSB_A)
_sc_call_b = _mk_sc_call(_NSSB_A, _NSSB_B)


def _tc_body(m_ref, b_ref,
             f0_ref, f1_ref, f2_ref, f3_ref,
             o0_ref, o1_ref, o2_ref, o3_ref):
    mw = m_ref[...]
    bias = b_ref[...]
    fs = (f0_ref, f1_ref, f2_ref, f3_ref)
    os = (o0_ref, o1_ref, o2_ref, o3_ref)
    v = jnp.zeros((RC, CH), jnp.float32)
    for t in range(T):
        feat = pltpu.bitcast(fs[t][...], jnp.bfloat16).reshape(RC, 2 * CH)
        conv = lax.dot_general(feat, mw, (((1,), (0,)), ((), ())),
                               preferred_element_type=jnp.float32) + bias
        v = v + conv
        spk = v >= 1.0
        s = jnp.where(spk, 1.0, 0.0)
        v = jnp.where(spk, 0.0, v)
        m = s.reshape(RC // K, K, CH)
        m = jnp.maximum(m[:, :8], m[:, 8:])
        m = jnp.maximum(m[:, :4], m[:, 4:])
        m = jnp.maximum(m[:, :2], m[:, 2:])
        m = jnp.maximum(m[:, :1], m[:, 1:])
        os[t][...] = m.reshape(RC // K, CH)


def _tc_call(mw, bias, feats):
    e_c = feats[0].shape[0]
    grid = (e_c // RC,)
    mspec = pl.BlockSpec((2 * CH, 128), lambda g: (0, 0))
    bspec = pl.BlockSpec((1, 128), lambda g: (0, 0))
    fspec = pl.BlockSpec((RC, 128), lambda g: (g, 0))
    ospec = pl.BlockSpec((RC // K, 128), lambda g: (g, 0))
    return pl.pallas_call(
        _tc_body,
        grid=grid,
        in_specs=[mspec, bspec] + [fspec] * 4,
        out_specs=[ospec] * 4,
        out_shape=[jax.ShapeDtypeStruct((e_c // K, CH), jnp.float32)] * 4,
    )(mw, bias, *feats)


@jax.jit
def kernel(x, edge_index, W, b):
    xf = x[..., 0]
    xfp = jnp.pad(xf.transpose(0, 2, 1), ((0, 0), (0, NPAD - N), (0, 0)))
    xfT = xfp.reshape(T * NPAD, CH)

    ei = jnp.pad(edge_index, ((0, 0), (0, 0), (0, NPAD - N), (0, 0)))
    i1 = ei[1].reshape(T * SBR, 128)
    i0 = ei[0].reshape(T * SBR, 128)

    Wg = W.reshape(4, 32, 64)
    mw = jnp.zeros((256, 128), jnp.float32)
    mw = mw.at[0:64, 0:32].set(Wg[0].T)
    mw = mw.at[64:128, 32:64].set(Wg[1].T)
    mw = mw.at[128:192, 64:96].set(Wg[2].T)
    mw = mw.at[192:256, 96:128].set(Wg[3].T)
    mw = mw.astype(jnp.bfloat16)
    bias = b[None, :]

    feats_a = _sc_call_a(xfT, i1, i0)
    outs_a = _tc_call(mw, bias, feats_a)
    feats_b = _sc_call_b(xfT, i1, i0)
    outs_b = _tc_call(mw, bias, feats_b)
    npt_a, npt_b = _NSSB_A * 64, _NSSB_B * 64
    oa = jnp.stack(outs_a).reshape(T, NW, npt_a, CH)
    ob = jnp.stack(outs_b).reshape(T, NW, npt_b, CH)
    out = jnp.concatenate([oa, ob], axis=2).reshape(T, NPAD, CH)[:, :N]
    return out.transpose(0, 2, 1)[:, None, :, :, None]
